# baseline (speedup 1.0000x reference)
import math

import jax
import jax.numpy as jnp
from jax import lax
from jax.experimental import pallas as pl
from jax.experimental.pallas import tpu as pltpu
from jax.experimental.pallas import tpu_sc as plsc

D = 64
NC = 2
NS = 16
NW = NC * NS
CH = 128
NBUF = 2
SCALE = math.sqrt(D)


def _sc_body(x_hbm, pe_hbm, table_hbm, out_hbm, idx_v, pe_v, in_v, ou_v, sg, sw):
    seq = x_hbm.shape[0]
    wid = lax.axis_index("s") * NC + lax.axis_index("c")

    pltpu.sync_copy(x_hbm.at[:, wid], idx_v)
    pltpu.sync_copy(pe_hbm, pe_v)

    def compute(c, buf):
        pev = [pe_v[c, pl.ds(16 * q, 16)] for q in range(D // 16)]
        def group_body(g, _):
            idx16 = idx_v[c, pl.ds(g * 16, 16)]
            scale16 = jnp.where(idx16 == 0, 0.0, SCALE)
            for jj in range(16):
                j = g * 16 + jj
                sv = jnp.full((16,), scale16[jj], jnp.float32)
                for q in range(D // 16):
                    row = in_v[buf, j, pl.ds(16 * q, 16)]
                    ou_v[buf, j, pl.ds(16 * q, 16)] = row * sv + pev[q]
            return 0
        lax.fori_loop(0, CH // 16, group_body, 0)

    def start_gather(c, buf):
        pltpu.async_copy(table_hbm.at[idx_v.at[c]], in_v.at[buf], sg.at[buf])

    def wait_gather(buf):
        pltpu.make_async_copy(table_hbm.at[pl.ds(0, CH)], in_v.at[buf],
                              sg.at[buf]).wait()

    def start_write(c, buf):
        pltpu.async_copy(ou_v.at[buf],
                         out_hbm.at[pl.ds(wid * CH, CH), c], sw.at[buf])

    def wait_write(buf):
        pltpu.make_async_copy(table_hbm.at[pl.ds(0, CH)], ou_v.at[buf],
                              sw.at[buf]).wait()

    for b in range(NBUF):
        start_gather(b, b)

    def ring_body(g, _):
        for b in range(NBUF):
            c = NBUF * g + b
            wait_gather(b)
            @pl.when(g >= 1)
            def _():
                wait_write(b)
            compute(c, b)
            @pl.when(g < seq // NBUF - 1)
            def _():
                start_gather(c + NBUF, b)
            start_write(c, b)
        return 0

    lax.fori_loop(0, seq // NBUF, ring_body, 0)
    for b in range(NBUF):
        wait_write(b)


@jax.jit
def kernel(X, table, pe):
    B, T = X.shape
    x3d = X.T.reshape(T, NW, CH)
    pe2 = pe[:T]
    tbl128 = jnp.pad(table, ((0, 0), (0, 128 - D)))

    mesh = plsc.VectorSubcoreMesh(core_axis_name="c", subcore_axis_name="s",
                                  num_cores=NC, num_subcores=NS)
    out = pl.kernel(
        _sc_body,
        out_type=jax.ShapeDtypeStruct((B, T, D), jnp.float32),
        mesh=mesh,
        compiler_params=pltpu.CompilerParams(use_tc_tiling_on_sc=True),
        scratch_types=[
            pltpu.VMEM((T, CH), jnp.int32),
            pltpu.VMEM((T, D), jnp.float32),
            pltpu.VMEM((NBUF, CH, 128), jnp.float32),
            pltpu.VMEM((NBUF, CH, D), jnp.float32),
            pltpu.SemaphoreType.DMA((NBUF,)),
            pltpu.SemaphoreType.DMA((NBUF,)),
        ],
    )(x3d, pe2, tbl128)
    return out


# speedup vs baseline: 1.6034x; 1.6034x over previous
import math

import jax
import jax.numpy as jnp
from jax import lax
from jax.experimental import pallas as pl
from jax.experimental.pallas import tpu as pltpu
from jax.experimental.pallas import tpu_sc as plsc

D = 64
NC = 2
NS = 16
NW = NC * NS
CH = 128
NBUF = 4
SCALE = math.sqrt(D)


def _sc_body(x_hbm, pe_hbm, table_hbm, out_hbm, idx_v, pe_v, in_v, ou_v, sg, sw):
    seq = x_hbm.shape[0]
    wid = lax.axis_index("s") * NC + lax.axis_index("c")

    pltpu.sync_copy(x_hbm.at[:, wid], idx_v)
    pltpu.sync_copy(pe_hbm, pe_v)

    def compute(c, buf):
        pev = [pe_v[c, pl.ds(16 * q, 16)] for q in range(D // 16)]
        def group_body(g, _):
            idx16 = idx_v[c, pl.ds(g * 16, 16)]
            scale16 = jnp.where(idx16 == 0, 0.0, SCALE)
            for jj in range(16):
                j = g * 16 + jj
                sv = jnp.full((16,), scale16[jj], jnp.float32)
                for q in range(D // 16):
                    row = in_v[buf, j, pl.ds(16 * q, 16)]
                    ou_v[buf, j, pl.ds(16 * q, 16)] = row * sv + pev[q]
            return 0
        lax.fori_loop(0, CH // 16, group_body, 0)

    def start_gather(c, buf):
        pltpu.async_copy(table_hbm.at[idx_v.at[c]], in_v.at[buf], sg.at[buf])

    def wait_gather(buf):
        pltpu.make_async_copy(table_hbm.at[pl.ds(0, CH)], in_v.at[buf],
                              sg.at[buf]).wait()

    def start_write(c, buf):
        pltpu.async_copy(ou_v.at[buf],
                         out_hbm.at[pl.ds(wid * CH, CH), c, pl.ds(0, D)],
                         sw.at[buf])

    def wait_write(buf):
        pltpu.make_async_copy(table_hbm.at[pl.ds(0, CH)], ou_v.at[buf],
                              sw.at[buf]).wait()

    for b in range(NBUF):
        start_gather(b, b)

    def ring_body(g, _):
        for b in range(NBUF):
            c = NBUF * g + b
            wait_gather(b)
            @pl.when(g >= 1)
            def _():
                wait_write(b)
            compute(c, b)
            @pl.when(g < seq // NBUF - 1)
            def _():
                start_gather(c + NBUF, b)
            start_write(c, b)
        return 0

    lax.fori_loop(0, seq // NBUF, ring_body, 0)
    for b in range(NBUF):
        wait_write(b)


@jax.jit
def kernel(X, table, pe):
    B, T = X.shape
    x3d = X.T.reshape(T, NW, CH)
    pe2 = pe[:T]

    mesh = plsc.VectorSubcoreMesh(core_axis_name="c", subcore_axis_name="s",
                                  num_cores=NC, num_subcores=NS)
    out = pl.kernel(
        _sc_body,
        out_type=jax.ShapeDtypeStruct((B, T, 128), jnp.float32),
        mesh=mesh,
        compiler_params=pltpu.CompilerParams(use_tc_tiling_on_sc=False),
        scratch_types=[
            pltpu.VMEM((T, CH), jnp.int32),
            pltpu.VMEM((T, D), jnp.float32),
            pltpu.VMEM((NBUF, CH, D), jnp.float32),
            pltpu.VMEM((NBUF, CH, D), jnp.float32),
            pltpu.SemaphoreType.DMA((NBUF,)),
            pltpu.SemaphoreType.DMA((NBUF,)),
        ],
    )(x3d, pe2, table)
    return out[:, :, :D]
